# Initial kernel scaffold; baseline (speedup 1.0000x reference)
import jax
import jax.numpy as jnp
from jax.experimental import pallas as pl


def kernel(x, router_w, w_gate_up, w_down):
    raise NotImplementedError("write your pallas kernel here")


# speedup vs baseline: 1.6180x; 1.6180x over previous
import jax
import jax.numpy as jnp
from jax import lax
from jax.experimental import pallas as pl
from jax.experimental.pallas import tpu as pltpu

D_MODEL = 1024
D_FF = 2048
N_EXPERTS = 8
N_TOKENS = 2048
EPS = 1e-8
CLAMP = 10000.0
FB = 1024
NF = D_FF // FB


def _router_combine(xv, rw):
    logits = lax.dot_general(xv, rw, (((1,), (1,)), ((), ())))
    logits = jnp.clip(logits, -CLAMP, CLAMP)
    m = jnp.max(logits, axis=-1, keepdims=True)
    e = jnp.exp(logits - m)
    p = e / (jnp.sum(e, axis=-1, keepdims=True) + EPS)
    p = jnp.clip(p, EPS, 1.0)
    iota = lax.broadcasted_iota(jnp.int32, p.shape, 1)
    m1 = jnp.max(p, axis=-1, keepdims=True)
    i1 = jnp.min(jnp.where(p == m1, iota, N_EXPERTS), axis=-1, keepdims=True)
    p2 = jnp.where(iota == i1, -1.0, p)
    m2 = jnp.max(p2, axis=-1, keepdims=True)
    i2 = jnp.min(jnp.where(p2 == m2, iota, N_EXPERTS), axis=-1, keepdims=True)
    denom = m1 + m2
    c = (jnp.where(iota == i1, m1, 0.0) + jnp.where(iota == i2, m2, 0.0)) / denom
    return c


def _body(x_ref, rw_ref, wg_ref, wu_ref, wd_ref, out_ref, c_scr):
    e = pl.program_id(0)
    f = pl.program_id(1)

    @pl.when((e == 0) & (f == 0))
    def _init():
        c_scr[...] = _router_combine(x_ref[...], rw_ref[...])
        out_ref[...] = jnp.zeros_like(out_ref)

    xv = x_ref[...]
    gate = lax.dot_general(xv, wg_ref[0, 0], (((1,), (1,)), ((), ())))
    up = lax.dot_general(xv, wu_ref[0, 0], (((1,), (1,)), ((), ())))
    h = (up * jax.nn.sigmoid(up)) * gate
    yp = lax.dot_general(h, wd_ref[0], (((1,), (1,)), ((), ())))
    iota = lax.broadcasted_iota(jnp.int32, (N_TOKENS, N_EXPERTS), 1)
    c_col = jnp.sum(jnp.where(iota == e, c_scr[...], 0.0), axis=1, keepdims=True)
    out_ref[...] += c_col * yp


def kernel(x, router_w, w_gate_up, w_down):
    wg2 = w_gate_up.reshape(N_EXPERTS, 2, D_FF, D_MODEL)
    return pl.pallas_call(
        _body,
        grid=(N_EXPERTS, NF),
        in_specs=[
            pl.BlockSpec((N_TOKENS, D_MODEL), lambda e, f: (0, 0)),
            pl.BlockSpec((N_EXPERTS, D_MODEL), lambda e, f: (0, 0)),
            pl.BlockSpec((1, 1, FB, D_MODEL), lambda e, f: (e, 0, f, 0)),
            pl.BlockSpec((1, 1, FB, D_MODEL), lambda e, f: (e, 1, f, 0)),
            pl.BlockSpec((1, D_MODEL, FB), lambda e, f: (e, 0, f)),
        ],
        out_specs=pl.BlockSpec((N_TOKENS, D_MODEL), lambda e, f: (0, 0)),
        out_shape=jax.ShapeDtypeStruct((N_TOKENS, D_MODEL), jnp.float32),
        scratch_shapes=[pltpu.VMEM((N_TOKENS, N_EXPERTS), jnp.float32)],
        compiler_params=pltpu.CompilerParams(
            dimension_semantics=("arbitrary", "arbitrary"),
        ),
    )(x, router_w, wg2, wg2, w_down)
